# baseline (speedup 1.0000x reference)
import functools

import jax
import jax.numpy as jnp
import numpy as np
from jax import lax
from jax.experimental import pallas as pl
from jax.experimental.pallas import tpu as pltpu
from jax.experimental.pallas import tpu_sc as plsc

N = 10000
E = 320000
D = 128
NC = 2
NS = 16
NT = NC * NS
CH = 128
CPT = 80
EPT = CPT * CH
ECH = E // CH
RCH31 = ECH - (NT - 1) * CPT
PCH = NT * CPT - ECH
NPAD = 10240
STRIPE = NPAD // NS

_PROW = (N + np.arange(PCH * CH) % (NPAD - N)).reshape(PCH, CH).astype(np.int32)
_PCOL = ((np.arange(PCH * CH) * 37) % N).reshape(PCH, CH).astype(np.int32)

_mesh = plsc.VectorSubcoreMesh(
    core_axis_name="c", subcore_axis_name="s", num_cores=NC, num_subcores=NS
)


HR = NPAD // CH


@functools.partial(
    pl.kernel,
    out_type=jax.ShapeDtypeStruct((NC, HR, CH), jnp.float32),
    mesh=_mesh,
    scratch_types=[
        pltpu.VMEM((CPT, CH), jnp.int32),
        pltpu.VMEM((HR, CH), jnp.float32),
        pltpu.VMEM((HR,), jnp.int32),
        pltpu.VMEM_SHARED((HR, CH), jnp.float32),
    ],
    compiler_params=pltpu.CompilerParams(needs_layout_passes=False),
)
def _degree_kernel(row_hbm, zeros_hbm, out_hbm, ridx_v, hist_v, riota_v, hist_sh):
    c = lax.axis_index("c")
    s = lax.axis_index("s")
    g = c * NS + s
    pltpu.sync_copy(row_hbm.at[pl.ds(g * CPT, CPT)], ridx_v)

    @pl.when(s == 0)
    def _():
        pltpu.sync_copy(zeros_hbm.at[pl.ds(0, HR), pl.ds(0, CH)], hist_sh)

    zero = jnp.zeros((16,), jnp.float32)
    lanes = lax.iota(jnp.int32, 16)
    for k in range(HR // 16):
        riota_v[pl.ds(k * 16, 16)] = lanes + (k * 16)

    def zbody(i, carry):
        for u in range(CH // 16):
            hist_v[i, pl.ds(u * 16, 16)] = zero
        return carry

    lax.fori_loop(0, HR, zbody, 0)

    def body(j, carry):
        for u in range(CH // 16):
            idx = ridx_v[j, pl.ds(u * 16, 16)]
            cnt, last = plsc.scan_count(idx)
            plsc.addupdate_scatter(
                hist_v,
                [lax.shift_right_logical(idx, 7), lax.bitwise_and(idx, 127)],
                cnt.astype(jnp.float32),
                mask=last,
            )
        return carry

    lax.fori_loop(0, CPT, body, 0)
    plsc.subcore_barrier()
    pltpu.sync_copy(hist_v, hist_sh.at[riota_v], add=True)
    plsc.subcore_barrier()

    @pl.when(s == 0)
    def _():
        pltpu.sync_copy(hist_sh, out_hbm.at[c])


def _prescale_body(h_ref, x_ref, y_ref):
    h = h_ref[...]
    deg = h[0:1] + h[1:2]
    safe = jnp.where(deg > 0, deg, 1.0)
    dinv = jnp.where(deg > 0, lax.rsqrt(safe), 0.0)
    dcol = jnp.transpose(dinv)
    y_ref[...] = x_ref[...] * dcol[:N]


@functools.partial(
    pl.kernel,
    out_type=jax.ShapeDtypeStruct((NC, NPAD, D), jnp.float32),
    mesh=_mesh,
    scratch_types=[
        pltpu.VMEM((CPT // 2, CH), jnp.int32),
        pltpu.VMEM((CPT // 2, CH), jnp.int32),
        pltpu.VMEM((CH, D), jnp.float32),
        pltpu.VMEM((CH, D), jnp.float32),
        pltpu.VMEM_SHARED((NPAD, D), jnp.float32),
        pltpu.SemaphoreType.DMA,
        pltpu.SemaphoreType.DMA,
    ],
)
def _spmm_kernel(row_hbm, col_hbm, y_hbm, zeros_hbm,
                 out_hbm, ridx_v, cidx_v, gbuf0, gbuf1, acc_sh, sem0, sem1):
    c = lax.axis_index("c")
    s = lax.axis_index("s")
    g = c * NS + s
    hcpt = CPT // 2
    pltpu.sync_copy(row_hbm.at[pl.ds(g * CPT, hcpt)], ridx_v)
    pltpu.sync_copy(col_hbm.at[pl.ds(g * CPT, hcpt)], cidx_v)
    pltpu.async_copy(y_hbm.at[cidx_v.at[0]], gbuf0, sem0)
    pltpu.sync_copy(
        zeros_hbm.at[pl.ds(s * STRIPE, STRIPE)],
        acc_sh.at[pl.ds(s * STRIPE, STRIPE)],
    )
    plsc.subcore_barrier()

    for ph in range(2):
        if ph:
            pltpu.sync_copy(row_hbm.at[pl.ds(g * CPT + ph * hcpt, hcpt)], ridx_v)
            pltpu.sync_copy(col_hbm.at[pl.ds(g * CPT + ph * hcpt, hcpt)], cidx_v)
            pltpu.async_copy(y_hbm.at[cidx_v.at[0]], gbuf0, sem0)

        def body(i, carry):
            j0 = 2 * i
            j1 = j0 + 1
            pltpu.async_copy(y_hbm.at[cidx_v.at[j1]], gbuf1, sem1)
            pltpu.make_async_copy(y_hbm.at[cidx_v.at[j0]], gbuf0, sem0).wait()
            pltpu.sync_copy(gbuf0, acc_sh.at[ridx_v.at[j0]], add=True)

            @pl.when(j0 + 2 < hcpt)
            def _():
                pltpu.async_copy(y_hbm.at[cidx_v.at[j0 + 2]], gbuf0, sem0)

            pltpu.make_async_copy(y_hbm.at[cidx_v.at[j1]], gbuf1, sem1).wait()
            pltpu.sync_copy(gbuf1, acc_sh.at[ridx_v.at[j1]], add=True)
            return carry

        lax.fori_loop(0, hcpt // 2, body, 0)
    plsc.subcore_barrier()
    pltpu.sync_copy(
        acc_sh.at[pl.ds(s * STRIPE, STRIPE)],
        out_hbm.at[c, pl.ds(s * STRIPE, STRIPE)],
    )


def _post_body(h_ref, p_ref, o_ref):
    h = h_ref[...]
    deg = h[0:1] + h[1:2]
    safe = jnp.where(deg > 0, deg, 1.0)
    dinv = jnp.where(deg > 0, lax.rsqrt(safe), 0.0)
    dcol = jnp.transpose(dinv)
    o_ref[...] = (p_ref[0, :N, :] + p_ref[1, :N, :]) * dcol[:N]


def kernel(features, edge_index):
    features = features.astype(jnp.float32)
    row2 = edge_index[0].astype(jnp.int32).reshape(ECH, CH)

    zeros = jnp.zeros((NPAD, D), jnp.float32)
    rowp = jnp.concatenate([row2, _PROW], axis=0)

    hist3 = _degree_kernel(rowp, zeros)
    hist = hist3.reshape(NC, NPAD)

    ei2, _ = lax.optimization_barrier((edge_index, rowp))
    colp = jnp.concatenate(
        [ei2[1].astype(jnp.int32).reshape(ECH, CH), _PCOL], axis=0
    )

    y = pl.pallas_call(
        _prescale_body,
        out_shape=jax.ShapeDtypeStruct((N, D), jnp.float32),
    )(hist, features)

    partials = _spmm_kernel(rowp, colp, y, zeros)

    out = pl.pallas_call(
        _post_body,
        out_shape=jax.ShapeDtypeStruct((N, D), jnp.float32),
    )(hist, partials)
    return out


# speedup vs baseline: 1.0793x; 1.0793x over previous
import functools

import jax
import jax.numpy as jnp
import numpy as np
from jax import lax
from jax.experimental import pallas as pl
from jax.experimental.pallas import tpu as pltpu
from jax.experimental.pallas import tpu_sc as plsc

N = 10000
E = 320000
D = 128
NC = 2
NS = 16
NT = NC * NS
CH = 128
CPT = 80
EPT = CPT * CH
ECH = E // CH
RCH31 = ECH - (NT - 1) * CPT
PCH = NT * CPT - ECH
NPAD = 10240
STRIPE = NPAD // NS

NPE = PCH * CH
R31 = E - (NT - 1) * EPT
_EPAD = np.stack([
    (N + np.arange(NPE) % (NPAD - N)).astype(np.int32),
    ((np.arange(NPE) * 37) % N).astype(np.int32),
])

_mesh = plsc.VectorSubcoreMesh(
    core_axis_name="c", subcore_axis_name="s", num_cores=NC, num_subcores=NS
)


HR = NPAD // CH


@functools.partial(
    pl.kernel,
    out_type=jax.ShapeDtypeStruct((NC, HR, CH), jnp.float32),
    mesh=_mesh,
    scratch_types=[
        pltpu.VMEM((2, EPT), jnp.int32),
        pltpu.VMEM((HR, CH), jnp.float32),
        pltpu.VMEM((HR,), jnp.int32),
        pltpu.VMEM((16, CH), jnp.float32),
        pltpu.VMEM_SHARED((HR, CH), jnp.float32),
    ],
    compiler_params=pltpu.CompilerParams(needs_layout_passes=False),
)
def _degree_kernel(ei_hbm, pad_hbm, out_hbm, eidx_v, hist_v, riota_v, zbuf_v,
                   hist_sh):
    c = lax.axis_index("c")
    s = lax.axis_index("s")
    g = c * NS + s

    @pl.when(g < NT - 1)
    def _():
        pltpu.sync_copy(ei_hbm.at[:, pl.ds(g * EPT, EPT)], eidx_v)

    @pl.when(g == NT - 1)
    def _():
        pltpu.sync_copy(
            ei_hbm.at[:, pl.ds((NT - 1) * EPT, R31)], eidx_v.at[:, pl.ds(0, R31)]
        )
        pltpu.sync_copy(pad_hbm, eidx_v.at[:, pl.ds(R31, NPE)])

    zero = jnp.zeros((16,), jnp.float32)
    lanes = lax.iota(jnp.int32, 16)
    for k in range(HR // 16):
        riota_v[pl.ds(k * 16, 16)] = lanes + (k * 16)

    @pl.when(s == 0)
    def _():
        def zsh(i, carry):
            for u in range(CH // 16):
                zbuf_v[i, pl.ds(u * 16, 16)] = zero
            return carry

        lax.fori_loop(0, 16, zsh, 0)
        for k in range(HR // 16):
            pltpu.sync_copy(zbuf_v, hist_sh.at[pl.ds(k * 16, 16)])

    def zbody(i, carry):
        for u in range(CH // 16):
            hist_v[i, pl.ds(u * 16, 16)] = zero
        return carry

    lax.fori_loop(0, HR, zbody, 0)

    def body(j, carry):
        for u in range(CH // 16):
            idx = eidx_v[0, pl.ds(j * CH + u * 16, 16)]
            cnt, last = plsc.scan_count(idx)
            plsc.addupdate_scatter(
                hist_v,
                [lax.shift_right_logical(idx, 7), lax.bitwise_and(idx, 127)],
                cnt.astype(jnp.float32),
                mask=last,
            )
        return carry

    lax.fori_loop(0, CPT, body, 0)
    plsc.subcore_barrier()
    pltpu.sync_copy(hist_v, hist_sh.at[riota_v], add=True)
    plsc.subcore_barrier()

    @pl.when(s == 0)
    def _():
        pltpu.sync_copy(hist_sh, out_hbm.at[c])


def _prescale_body(h_ref, x_ref, y_ref):
    h = h_ref[...]
    deg = h[0:1] + h[1:2]
    safe = jnp.where(deg > 0, deg, 1.0)
    dinv = jnp.where(deg > 0, lax.rsqrt(safe), 0.0)
    dcol = jnp.transpose(dinv)
    y_ref[...] = x_ref[...] * dcol[:N]


@functools.partial(
    pl.kernel,
    out_type=jax.ShapeDtypeStruct((NC, NPAD, D), jnp.float32),
    mesh=_mesh,
    scratch_types=[
        pltpu.VMEM((2, CPT // 2 * CH), jnp.int32),
        pltpu.VMEM((CPT // 2, CH), jnp.int32),
        pltpu.VMEM((CH, D), jnp.float32),
        pltpu.VMEM((CH, D), jnp.float32),
        pltpu.VMEM_SHARED((NPAD, D), jnp.float32),
        pltpu.SemaphoreType.DMA,
        pltpu.SemaphoreType.DMA,
    ],
)
def _spmm_kernel(ei_hbm, pad_hbm, y_hbm, zeros_hbm,
                 out_hbm, eidx_v, ridx_v, gbuf0, gbuf1, acc_sh, sem0, sem1):
    c = lax.axis_index("c")
    s = lax.axis_index("s")
    g = c * NS + s
    hcpt = CPT // 2
    hept = hcpt * CH

    def stage(ph):
        @pl.when(g < NT - 1)
        def _():
            pltpu.sync_copy(ei_hbm.at[:, pl.ds(g * EPT + ph * hept, hept)], eidx_v)

        @pl.when(g == NT - 1)
        def _():
            if ph == 0:
                pltpu.sync_copy(
                    ei_hbm.at[:, pl.ds((NT - 1) * EPT, R31)],
                    eidx_v.at[:, pl.ds(0, R31)],
                )
                pltpu.sync_copy(
                    pad_hbm.at[:, pl.ds(0, hept - R31)],
                    eidx_v.at[:, pl.ds(R31, hept - R31)],
                )
            else:
                pltpu.sync_copy(pad_hbm.at[:, pl.ds(hept - R31, hept)], eidx_v)

        def rbody(j, carry):
            for u in range(CH // 16):
                ridx_v[j, pl.ds(u * 16, 16)] = eidx_v[0, pl.ds(j * CH + u * 16, 16)]
            return carry

        lax.fori_loop(0, hcpt, rbody, 0)

    def cidx(j):
        return eidx_v.at[1, pl.ds(j * CH, CH)]

    stage(0)
    pltpu.async_copy(y_hbm.at[cidx(0)], gbuf0, sem0)
    pltpu.sync_copy(
        zeros_hbm.at[pl.ds(s * STRIPE, STRIPE)],
        acc_sh.at[pl.ds(s * STRIPE, STRIPE)],
    )
    plsc.subcore_barrier()

    for ph in range(2):
        if ph:
            stage(ph)
            pltpu.async_copy(y_hbm.at[cidx(0)], gbuf0, sem0)

        def body(i, carry):
            j0 = 2 * i
            j1 = j0 + 1
            pltpu.async_copy(y_hbm.at[cidx(j1)], gbuf1, sem1)
            pltpu.make_async_copy(y_hbm.at[cidx(j0)], gbuf0, sem0).wait()
            pltpu.sync_copy(gbuf0, acc_sh.at[ridx_v.at[j0]], add=True)

            @pl.when(j0 + 2 < hcpt)
            def _():
                pltpu.async_copy(y_hbm.at[cidx(j0 + 2)], gbuf0, sem0)

            pltpu.make_async_copy(y_hbm.at[cidx(j1)], gbuf1, sem1).wait()
            pltpu.sync_copy(gbuf1, acc_sh.at[ridx_v.at[j1]], add=True)
            return carry

        lax.fori_loop(0, hcpt // 2, body, 0)
    plsc.subcore_barrier()
    pltpu.sync_copy(
        acc_sh.at[pl.ds(s * STRIPE, STRIPE)],
        out_hbm.at[c, pl.ds(s * STRIPE, STRIPE)],
    )


def _post_body(h_ref, p_ref, o_ref):
    h = h_ref[...]
    deg = h[0:1] + h[1:2]
    safe = jnp.where(deg > 0, deg, 1.0)
    dinv = jnp.where(deg > 0, lax.rsqrt(safe), 0.0)
    dcol = jnp.transpose(dinv)
    o_ref[...] = (p_ref[0, :N, :] + p_ref[1, :N, :]) * dcol[:N]


def kernel(features, edge_index):
    features = features.astype(jnp.float32)
    ei = edge_index.astype(jnp.int32)
    zeros = jnp.zeros((NPAD, D), jnp.float32)

    hist3 = _degree_kernel(ei, _EPAD)
    hist = hist3.reshape(NC, NPAD)

    y = pl.pallas_call(
        _prescale_body,
        out_shape=jax.ShapeDtypeStruct((N, D), jnp.float32),
    )(hist, features)

    partials = _spmm_kernel(ei, _EPAD, y, zeros)

    out = pl.pallas_call(
        _post_body,
        out_shape=jax.ShapeDtypeStruct((N, D), jnp.float32),
    )(hist, partials)
    return out
